# baseline (speedup 1.0000x reference)
import jax
import jax.numpy as jnp
from jax import lax
from jax.experimental import pallas as pl
from jax.experimental.pallas import tpu as pltpu
from jax.experimental.pallas import tpu_sc as plsc

NC = 2
NS = 16
L = 16
NW = NC * NS

K = 3
D = 1024
CH = 8
NB = 4

UNROLL = 1


def _body(table_hbm, ids_hbm, logits_hbm, out_hbm,
          idx_v, logits_v, weights_v, rows_v, out_v,
          gsem0, gsem1, gsem2, gsem3, osem0, osem1):
    ntok = out_hbm.shape[0]
    tpw = ntok // NW
    nch = tpw // CH
    gsem = (gsem0, gsem1, gsem2, gsem3)
    osem = (osem0, osem1)

    cid = lax.axis_index("c")
    sid = lax.axis_index("s")
    wid = sid * NC + cid
    tok0 = wid * tpw

    pltpu.sync_copy(ids_hbm.at[pl.ds(wid * nch, nch)], idx_v)
    for p in range(NB - 1):
        pltpu.async_copy(table_hbm.at[idx_v.at[p]], rows_v.at[p], gsem[p])
    pltpu.sync_copy(logits_hbm.at[pl.ds(wid * tpw * K, tpw * K)], logits_v)

    for tg in range(tpw // L):
        tvec = jnp.arange(L, dtype=jnp.int32) * K + (tg * L * K)
        w0 = plsc.load_gather(logits_v, [tvec])
        w1 = plsc.load_gather(logits_v, [tvec + 1])
        w2 = plsc.load_gather(logits_v, [tvec + 2])
        m = jnp.maximum(w0, jnp.maximum(w1, w2))
        e0 = jnp.exp(w0 - m)
        e1 = jnp.exp(w1 - m)
        e2 = jnp.exp(w2 - m)
        inv = 1.0 / (e0 + e1 + e2)
        weights_v[pl.ds(0 * tpw + tg * L, L)] = e0 * inv
        weights_v[pl.ds(1 * tpw + tg * L, L)] = e1 * inv
        weights_v[pl.ds(2 * tpw + tg * L, L)] = e2 * inv

    def ring_body(cq, carry):
        for b in range(NB):
            c = NB * cq + b
            pb = (b + NB - 1) % NB

            @pl.when(c + NB - 1 < nch)
            def _():
                pltpu.async_copy(
                    table_hbm.at[idx_v.at[c + NB - 1]],
                    rows_v.at[pb], gsem[pb])

            pltpu.make_async_copy(
                table_hbm.at[idx_v.at[c]], rows_v.at[b], gsem[b]).wait()

            ob = b % 2
            if b >= 2:
                pltpu.make_async_copy(
                    out_v.at[ob], out_hbm.at[pl.ds(tok0, CH)], osem[ob]).wait()
            else:
                @pl.when(cq > 0)
                def _():
                    pltpu.make_async_copy(
                        out_v.at[ob],
                        out_hbm.at[pl.ds(tok0, CH)], osem[ob]).wait()

            for t in range(CH):
                tix = c * CH + t
                w0, w1, w2 = (
                    plsc.load_gather(
                        weights_v,
                        [jnp.full((L,), kk * tpw + tix, jnp.int32)])
                    for kk in range(K))

                def d_body(d, t=t, w0=w0, w1=w1, w2=w2):
                    sl = pl.ds(d * L, L)
                    r0 = rows_v[b, K * t + 0, sl]
                    r1 = rows_v[b, K * t + 1, sl]
                    r2 = rows_v[b, K * t + 2, sl]
                    out_v[ob, t, sl] = w0 * r0 + w1 * r1 + w2 * r2

                plsc.parallel_loop(0, D // L, unroll=UNROLL)(d_body)

            pltpu.async_copy(
                out_v.at[ob], out_hbm.at[pl.ds(tok0 + c * CH, CH)], osem[ob])
        return carry

    lax.fori_loop(0, nch // NB, ring_body, 0)

    pltpu.make_async_copy(
        out_v.at[0], out_hbm.at[pl.ds(tok0, CH)], osem[0]).wait()
    pltpu.make_async_copy(
        out_v.at[1], out_hbm.at[pl.ds(tok0, CH)], osem[1]).wait()


def kernel(vocab_embeddings, random_ids, weight_logits):
    B, S, k = random_ids.shape
    ntok = B * S
    assert k == K and vocab_embeddings.shape[1] == D
    assert ntok % (NW * CH * NB) == 0
    tpw = ntok // NW
    nch = tpw // CH

    ids2d = random_ids.reshape(ntok // CH, CH * K)
    logits_flat = weight_logits.reshape(ntok * K)

    mesh = plsc.VectorSubcoreMesh(
        core_axis_name="c", subcore_axis_name="s",
        num_cores=NC, num_subcores=NS)

    run = pl.kernel(
        _body,
        out_type=jax.ShapeDtypeStruct((ntok, D), jnp.float32),
        mesh=mesh,
        scratch_types=[
            pltpu.VMEM((nch, CH * K), jnp.int32),
            pltpu.VMEM((tpw * K,), jnp.float32),
            pltpu.VMEM((K * tpw,), jnp.float32),
            pltpu.VMEM((NB, CH * K, D), jnp.float32),
            pltpu.VMEM((2, CH, D), jnp.float32),
            pltpu.SemaphoreType.DMA,
            pltpu.SemaphoreType.DMA,
            pltpu.SemaphoreType.DMA,
            pltpu.SemaphoreType.DMA,
            pltpu.SemaphoreType.DMA,
            pltpu.SemaphoreType.DMA,
        ],
        compiler_params=pltpu.CompilerParams(needs_layout_passes=False),
    )
    out = run(vocab_embeddings, ids2d, logits_flat)
    return out.reshape(B, S, D)


# speedup vs baseline: 1.2004x; 1.2004x over previous
import jax
import jax.numpy as jnp
from jax import lax
from jax.experimental import pallas as pl
from jax.experimental.pallas import tpu as pltpu
from jax.experimental.pallas import tpu_sc as plsc

NC = 2
NS = 16
L = 16
NW = NC * NS

K = 3
D = 1024
CH = 8
NB = 4

UNROLL = 2


def _body(table_hbm, ids_hbm, logits_hbm, out_hbm,
          idx_v, logits_v, weights_v, rows_v, out_v,
          gsem0, gsem1, gsem2, gsem3, osem0, osem1):
    ntok = out_hbm.shape[0]
    tpw = ntok // NW
    nch = tpw // CH
    gsem = (gsem0, gsem1, gsem2, gsem3)
    osem = (osem0, osem1)

    cid = lax.axis_index("c")
    sid = lax.axis_index("s")
    wid = sid * NC + cid
    tok0 = wid * tpw

    pltpu.sync_copy(ids_hbm.at[pl.ds(wid * nch, nch)], idx_v)
    for p in range(NB - 1):
        pltpu.async_copy(table_hbm.at[idx_v.at[p]], rows_v.at[p], gsem[p])
    pltpu.sync_copy(logits_hbm.at[pl.ds(wid * tpw * K, tpw * K)], logits_v)

    for tg in range(tpw // L):
        tvec = jnp.arange(L, dtype=jnp.int32) * K + (tg * L * K)
        w0 = plsc.load_gather(logits_v, [tvec])
        w1 = plsc.load_gather(logits_v, [tvec + 1])
        w2 = plsc.load_gather(logits_v, [tvec + 2])
        m = jnp.maximum(w0, jnp.maximum(w1, w2))
        e0 = jnp.exp(w0 - m)
        e1 = jnp.exp(w1 - m)
        e2 = jnp.exp(w2 - m)
        inv = 1.0 / (e0 + e1 + e2)
        weights_v[pl.ds(0 * tpw + tg * L, L)] = e0 * inv
        weights_v[pl.ds(1 * tpw + tg * L, L)] = e1 * inv
        weights_v[pl.ds(2 * tpw + tg * L, L)] = e2 * inv

    def ring_body(cq, carry):
        for b in range(NB):
            c = NB * cq + b
            pb = (b + NB - 1) % NB

            @pl.when(c + NB - 1 < nch)
            def _():
                pltpu.async_copy(
                    table_hbm.at[idx_v.at[c + NB - 1]],
                    rows_v.at[pb], gsem[pb])

            pltpu.make_async_copy(
                table_hbm.at[idx_v.at[c]], rows_v.at[b], gsem[b]).wait()

            ob = b % 2
            if b >= 2:
                pltpu.make_async_copy(
                    out_v.at[ob], out_hbm.at[pl.ds(tok0, CH)], osem[ob]).wait()
            else:
                @pl.when(cq > 0)
                def _():
                    pltpu.make_async_copy(
                        out_v.at[ob],
                        out_hbm.at[pl.ds(tok0, CH)], osem[ob]).wait()

            for t in range(CH):
                tix = c * CH + t
                w0, w1, w2 = (
                    plsc.load_gather(
                        weights_v,
                        [jnp.full((L,), kk * tpw + tix, jnp.int32)])
                    for kk in range(K))

                def d_body(d, t=t, w0=w0, w1=w1, w2=w2):
                    sl = pl.ds(d * L, L)
                    r0 = rows_v[b, K * t + 0, sl]
                    r1 = rows_v[b, K * t + 1, sl]
                    r2 = rows_v[b, K * t + 2, sl]
                    out_v[ob, t, sl] = w0 * r0 + w1 * r1 + w2 * r2

                plsc.parallel_loop(0, D // L, unroll=UNROLL)(d_body)

            pltpu.async_copy(
                out_v.at[ob], out_hbm.at[pl.ds(tok0 + c * CH, CH)], osem[ob])
        return carry

    lax.fori_loop(0, nch // NB, ring_body, 0)

    pltpu.make_async_copy(
        out_v.at[0], out_hbm.at[pl.ds(tok0, CH)], osem[0]).wait()
    pltpu.make_async_copy(
        out_v.at[1], out_hbm.at[pl.ds(tok0, CH)], osem[1]).wait()


def kernel(vocab_embeddings, random_ids, weight_logits):
    B, S, k = random_ids.shape
    ntok = B * S
    assert k == K and vocab_embeddings.shape[1] == D
    assert ntok % (NW * CH * NB) == 0
    tpw = ntok // NW
    nch = tpw // CH

    ids2d = random_ids.reshape(ntok // CH, CH * K)
    logits_flat = weight_logits.reshape(ntok * K)

    mesh = plsc.VectorSubcoreMesh(
        core_axis_name="c", subcore_axis_name="s",
        num_cores=NC, num_subcores=NS)

    run = pl.kernel(
        _body,
        out_type=jax.ShapeDtypeStruct((ntok, D), jnp.float32),
        mesh=mesh,
        scratch_types=[
            pltpu.VMEM((nch, CH * K), jnp.int32),
            pltpu.VMEM((tpw * K,), jnp.float32),
            pltpu.VMEM((K * tpw,), jnp.float32),
            pltpu.VMEM((NB, CH * K, D), jnp.float32),
            pltpu.VMEM((2, CH, D), jnp.float32),
            pltpu.SemaphoreType.DMA,
            pltpu.SemaphoreType.DMA,
            pltpu.SemaphoreType.DMA,
            pltpu.SemaphoreType.DMA,
            pltpu.SemaphoreType.DMA,
            pltpu.SemaphoreType.DMA,
        ],
        compiler_params=pltpu.CompilerParams(needs_layout_passes=False),
    )
    out = run(vocab_embeddings, ids2d, logits_flat)
    return out.reshape(B, S, D)
